# baseline (speedup 1.0000x reference)
import functools

import jax
import jax.numpy as jnp
from jax import lax
from jax.experimental import pallas as pl
from jax.experimental.pallas import tpu as pltpu
from jax.experimental.pallas import tpu_sc as plsc

VOCAB = 100000
BATCH = 1024
SEQ = 200
NTOK = BATCH * SEQ
N_WORKERS = 32
STRIPE = 3128
CHUNK_ROWS = 32
CHUNK_WORDS = CHUNK_ROWS * BATCH
TAIL_ROWS = 24
NBINS = 98
CAP = 16
NHALF = 2
LANES = 16
SLAB = 4096
N_SLABS = NTOK // SLAB
GROUPS_PER_SLAB = SLAB // LANES


def _sc_body(tok_hbm, out_hbm, slab_v, buf_v, list_v, cnt_v, dma_sem):
    w = lax.axis_index("s") * 2 + lax.axis_index("c")
    base = w * STRIPE
    size = jnp.minimum(STRIPE, VOCAB - base)
    base_code = base * BATCH
    lo = jnp.maximum(base_code, BATCH)
    hi = base_code + size * BATCH

    lane = lax.iota(jnp.int32, LANES)
    lane_eq = [lane == j for j in range(LANES)]
    zero_i = jnp.zeros((LANES,), jnp.int32)
    zero_f = jnp.zeros((LANES,), jnp.float32)
    one_i = jnp.ones((LANES,), jnp.int32)

    def zbuf(i, c):
        buf_v[i >> 11, (i >> 6) & 31, pl.ds((i & 63) * LANES, LANES)] = zero_f
        return c

    lax.fori_loop(0, 2 * CHUNK_WORDS // LANES, zbuf, 0)

    def zcnt(i, c):
        cnt_v[pl.ds(i * LANES, LANES)] = zero_i
        return c

    lax.fori_loop(0, NHALF * NBINS, zcnt, 0)

    def bin_one(g, h):
        tok = slab_v[pl.ds(g * LANES, LANES)]
        b = (g * LANES + lane) & (BATCH - 1)
        enc = (tok << 10) | b
        m = (enc >= lo) & (enc < hi)
        el = enc - base_code
        bin_ = jnp.where(m, el >> 15, 0)
        cidx = (h * NBINS + bin_) * LANES + lane
        pos = plsc.load_gather(cnt_v, [cidx], mask=m)
        inb = pos < CAP
        sm = m & inb
        plsc.store_scatter(
            list_v,
            [(h * NBINS + bin_) * (CAP * LANES) + pos * LANES + lane],
            el,
            mask=sm,
        )
        plsc.addupdate_scatter(cnt_v, [cidx], one_i, mask=sm)
        return m & ~inb

    def scan_group(g, ovf):
        return ovf | bin_one(2 * g, 0) | bin_one(2 * g + 1, 1)

    def scan_slab(s, ovf):
        pltpu.sync_copy(tok_hbm.at[pl.ds(s * SLAB, SLAB)], slab_v)
        return lax.fori_loop(0, GROUPS_PER_SLAB // 2, scan_group, ovf)

    ovf = lax.fori_loop(0, N_SLABS, scan_slab, lane < 0)
    any_ovf = jnp.max(ovf.astype(jnp.int32))

    n_full = size >> 5

    def scatter_chunk(c, p, value):
        val = jnp.full((LANES,), value, jnp.float32)
        pv = jnp.full((LANES,), 0, jnp.int32) + p
        chunk_code = c * CHUNK_WORDS

        for h in range(NHALF):
            cntc = cnt_v[pl.ds((h * NBINS + c) * LANES, LANES)]
            mx = jnp.max(cntc)

            def row_body(q, carry):
                row = list_v[
                    pl.ds((h * NBINS + c) * (CAP * LANES) + q * LANES, LANES)
                ]
                valid = q < cntc
                loc = row - chunk_code
                iv = loc >> 10
                ib = loc & (BATCH - 1)
                for j in range(LANES):
                    plsc.addupdate_scatter(
                        buf_v, [pv, iv, ib], val, mask=valid & lane_eq[j]
                    )
                return carry

            lax.fori_loop(0, mx, row_body, 0)

    def start_dma(c, p):
        pltpu.async_copy(
            buf_v.at[p],
            out_hbm.at[pl.ds(base + c * CHUNK_ROWS, CHUNK_ROWS)],
            dma_sem,
        )

    def wait_dma(c, p):
        pltpu.make_async_copy(
            buf_v.at[p],
            out_hbm.at[pl.ds(base + c * CHUNK_ROWS, CHUNK_ROWS)],
            dma_sem,
        ).wait()

    def tail_chunk(p):
        c = n_full
        scatter_chunk(c, p, 1.0)
        pltpu.sync_copy(
            buf_v.at[p, pl.ds(0, TAIL_ROWS)],
            out_hbm.at[pl.ds(base + c * CHUNK_ROWS, TAIL_ROWS)],
        )
        scatter_chunk(c, p, -1.0)

    @pl.when(any_ovf == 0)
    def _fast():
        def body(c, carry):
            p = c & 1

            @pl.when(c >= 2)
            def _():
                wait_dma(c - 2, p)
                scatter_chunk(c - 2, p, -1.0)

            scatter_chunk(c, p, 1.0)
            start_dma(c, p)
            return carry

        lax.fori_loop(0, n_full, body, 0)
        wait_dma(n_full - 2, n_full & 1)
        scatter_chunk(n_full - 2, n_full & 1, -1.0)
        wait_dma(n_full - 1, 1 - (n_full & 1))
        scatter_chunk(n_full - 1, 1 - (n_full & 1), -1.0)
        tail_chunk(0)

    def slow_chunk(c, rows):
        chunk_lo = jnp.maximum(base_code + c * CHUNK_WORDS, BATCH)
        chunk_hi = base_code + c * CHUNK_WORDS + rows * BATCH
        onef = jnp.full((LANES,), 1.0, jnp.float32)

        def sg(g, carry):
            tok = slab_v[pl.ds(g * LANES, LANES)]
            b = (g * LANES + lane) & (BATCH - 1)
            enc = (tok << 10) | b
            m = (enc >= chunk_lo) & (enc < chunk_hi)
            loc = enc - (base_code + c * CHUNK_WORDS)
            iv = jnp.where(m, loc >> 10, 0)
            ib = loc & (BATCH - 1)
            for j in range(LANES):
                plsc.addupdate_scatter(
                    buf_v, [zero_i, iv, ib], onef, mask=m & lane_eq[j]
                )
            return carry

        def ss(s, carry):
            pltpu.sync_copy(tok_hbm.at[pl.ds(s * SLAB, SLAB)], slab_v)
            return lax.fori_loop(0, GROUPS_PER_SLAB, sg, carry)

        lax.fori_loop(0, N_SLABS, ss, 0)
        pltpu.sync_copy(
            buf_v.at[0, pl.ds(0, rows)],
            out_hbm.at[pl.ds(base + c * CHUNK_ROWS, rows)],
        )

        def zb(i, carry):
            buf_v[0, i >> 6, pl.ds((i & 63) * LANES, LANES)] = zero_f
            return carry

        lax.fori_loop(0, rows * BATCH // LANES, zb, 0)

    @pl.when(any_ovf != 0)
    def _slow():
        def body(c, carry):
            slow_chunk(c, CHUNK_ROWS)
            return carry

        lax.fori_loop(0, n_full, body, 0)
        slow_chunk(n_full, TAIL_ROWS)


@jax.jit
def _encode(tokens):
    tok_flat = tokens.T.reshape(-1)
    mesh = plsc.VectorSubcoreMesh(core_axis_name="c", subcore_axis_name="s")
    out_t = pl.kernel(
        _sc_body,
        out_type=jax.ShapeDtypeStruct((VOCAB, BATCH), jnp.float32),
        mesh=mesh,
        compiler_params=pltpu.CompilerParams(needs_layout_passes=False),
        scratch_types=[
            pltpu.VMEM((SLAB,), jnp.int32),
            pltpu.VMEM((2, CHUNK_ROWS, BATCH), jnp.float32),
            pltpu.VMEM((NHALF * NBINS * CAP * LANES,), jnp.int32),
            pltpu.VMEM((NHALF * NBINS * LANES,), jnp.int32),
            pltpu.SemaphoreType.DMA,
        ],
    )(tok_flat)
    return out_t.T


def kernel(tokens, lengths):
    del lengths
    return _encode(tokens)


# speedup vs baseline: 1.0656x; 1.0656x over previous
import functools

import jax
import jax.numpy as jnp
from jax import lax
from jax.experimental import pallas as pl
from jax.experimental.pallas import tpu as pltpu
from jax.experimental.pallas import tpu_sc as plsc

VOCAB = 100000
BATCH = 1024
SEQ = 200
NTOK = BATCH * SEQ
N_WORKERS = 32
STRIPE = 3128
CHUNK_ROWS = 32
CHUNK_WORDS = CHUNK_ROWS * BATCH
TAIL_ROWS = 24
NBINS = 98
CAP = 16
NHALF = 2
LANES = 16
SLAB = 4096
N_SLABS = NTOK // SLAB
GROUPS_PER_SLAB = SLAB // LANES


def _sc_body(tok_hbm, out_hbm, slab_v, buf_v, list_v, cnt_v, dma_sem, slab_sem):
    w = lax.axis_index("s") * 2 + lax.axis_index("c")
    base = w * STRIPE
    size = jnp.minimum(STRIPE, VOCAB - base)
    base_code = base * BATCH
    lo = jnp.maximum(base_code, BATCH)
    hi = base_code + size * BATCH

    lane = lax.iota(jnp.int32, LANES)
    lane_eq = [lane == j for j in range(LANES)]
    zero_i = jnp.zeros((LANES,), jnp.int32)
    zero_f = jnp.zeros((LANES,), jnp.float32)
    one_i = jnp.ones((LANES,), jnp.int32)

    def zbuf(i, c):
        buf_v[i >> 11, (i >> 6) & 31, pl.ds((i & 63) * LANES, LANES)] = zero_f
        return c

    lax.fori_loop(0, 2 * CHUNK_WORDS // LANES, zbuf, 0)

    def zcnt(i, c):
        cnt_v[pl.ds(i * LANES, LANES)] = zero_i
        return c

    lax.fori_loop(0, NHALF * NBINS, zcnt, 0)

    def bin_one(sp, g, h):
        tok = slab_v[sp, pl.ds(g * LANES, LANES)]
        b = (g * LANES + lane) & (BATCH - 1)
        enc = (tok << 10) | b
        m = (enc >= lo) & (enc < hi)
        el = enc - base_code
        bin_ = jnp.where(m, el >> 15, 0)
        cidx = (h * NBINS + bin_) * LANES + lane
        pos = plsc.load_gather(cnt_v, [cidx], mask=m)
        inb = pos < CAP
        sm = m & inb
        plsc.store_scatter(
            list_v,
            [(h * NBINS + bin_) * (CAP * LANES) + pos * LANES + lane],
            el,
            mask=sm,
        )
        plsc.addupdate_scatter(cnt_v, [cidx], one_i, mask=sm)
        return m & ~inb

    def start_slab(s, sp):
        pltpu.async_copy(tok_hbm.at[pl.ds(s * SLAB, SLAB)], slab_v.at[sp], slab_sem)

    def wait_slab(s, sp):
        pltpu.make_async_copy(
            tok_hbm.at[pl.ds(s * SLAB, SLAB)], slab_v.at[sp], slab_sem
        ).wait()

    def scan_slab(s, ovf):
        sp = s & 1
        wait_slab(s, sp)

        @pl.when(s + 1 < N_SLABS)
        def _():
            start_slab(s + 1, 1 - sp)

        def scan_group(g, o):
            return o | bin_one(sp, 2 * g, 0) | bin_one(sp, 2 * g + 1, 1)

        return lax.fori_loop(0, GROUPS_PER_SLAB // 2, scan_group, ovf)

    start_slab(0, 0)
    ovf = lax.fori_loop(0, N_SLABS, scan_slab, lane < 0)
    any_ovf = jnp.max(ovf.astype(jnp.int32))

    n_full = size >> 5

    def scatter_chunk(c, p, value):
        val = jnp.full((LANES,), value, jnp.float32)
        pv = jnp.full((LANES,), 0, jnp.int32) + p
        chunk_code = c * CHUNK_WORDS

        for h in range(NHALF):
            cntc = cnt_v[pl.ds((h * NBINS + c) * LANES, LANES)]
            mx = jnp.max(cntc)

            def row_body(q, carry):
                row = list_v[
                    pl.ds((h * NBINS + c) * (CAP * LANES) + q * LANES, LANES)
                ]
                valid = q < cntc
                loc = row - chunk_code
                iv = loc >> 10
                ib = loc & (BATCH - 1)
                for j in range(LANES):
                    plsc.addupdate_scatter(
                        buf_v, [pv, iv, ib], val, mask=valid & lane_eq[j]
                    )
                return carry

            lax.fori_loop(0, mx, row_body, 0)

    def start_dma(c, p):
        pltpu.async_copy(
            buf_v.at[p],
            out_hbm.at[pl.ds(base + c * CHUNK_ROWS, CHUNK_ROWS)],
            dma_sem,
        )

    def wait_dma(c, p):
        pltpu.make_async_copy(
            buf_v.at[p],
            out_hbm.at[pl.ds(base + c * CHUNK_ROWS, CHUNK_ROWS)],
            dma_sem,
        ).wait()

    def tail_chunk(p):
        c = n_full
        scatter_chunk(c, p, 1.0)
        pltpu.sync_copy(
            buf_v.at[p, pl.ds(0, TAIL_ROWS)],
            out_hbm.at[pl.ds(base + c * CHUNK_ROWS, TAIL_ROWS)],
        )
        scatter_chunk(c, p, -1.0)

    @pl.when(any_ovf == 0)
    def _fast():
        def body(c, carry):
            p = c & 1

            @pl.when(c >= 2)
            def _():
                wait_dma(c - 2, p)
                scatter_chunk(c - 2, p, -1.0)

            scatter_chunk(c, p, 1.0)
            start_dma(c, p)
            return carry

        lax.fori_loop(0, n_full, body, 0)
        wait_dma(n_full - 2, n_full & 1)
        scatter_chunk(n_full - 2, n_full & 1, -1.0)
        wait_dma(n_full - 1, 1 - (n_full & 1))
        scatter_chunk(n_full - 1, 1 - (n_full & 1), -1.0)
        tail_chunk(0)

    def slow_chunk(c, rows):
        chunk_lo = jnp.maximum(base_code + c * CHUNK_WORDS, BATCH)
        chunk_hi = base_code + c * CHUNK_WORDS + rows * BATCH
        onef = jnp.full((LANES,), 1.0, jnp.float32)

        def sg(g, carry):
            tok = slab_v[0, pl.ds(g * LANES, LANES)]
            b = (g * LANES + lane) & (BATCH - 1)
            enc = (tok << 10) | b
            m = (enc >= chunk_lo) & (enc < chunk_hi)
            loc = enc - (base_code + c * CHUNK_WORDS)
            iv = jnp.where(m, loc >> 10, 0)
            ib = loc & (BATCH - 1)
            for j in range(LANES):
                plsc.addupdate_scatter(
                    buf_v, [zero_i, iv, ib], onef, mask=m & lane_eq[j]
                )
            return carry

        def ss(s, carry):
            pltpu.sync_copy(tok_hbm.at[pl.ds(s * SLAB, SLAB)], slab_v.at[0])
            return lax.fori_loop(0, GROUPS_PER_SLAB, sg, carry)

        lax.fori_loop(0, N_SLABS, ss, 0)
        pltpu.sync_copy(
            buf_v.at[0, pl.ds(0, rows)],
            out_hbm.at[pl.ds(base + c * CHUNK_ROWS, rows)],
        )

        def zb(i, carry):
            buf_v[0, i >> 6, pl.ds((i & 63) * LANES, LANES)] = zero_f
            return carry

        lax.fori_loop(0, rows * BATCH // LANES, zb, 0)

    @pl.when(any_ovf != 0)
    def _slow():
        def body(c, carry):
            slow_chunk(c, CHUNK_ROWS)
            return carry

        lax.fori_loop(0, n_full, body, 0)
        slow_chunk(n_full, TAIL_ROWS)


@jax.jit
def _encode(tokens):
    tok_flat = tokens.T.reshape(-1)
    mesh = plsc.VectorSubcoreMesh(core_axis_name="c", subcore_axis_name="s")
    out_t = pl.kernel(
        _sc_body,
        out_type=jax.ShapeDtypeStruct((VOCAB, BATCH), jnp.float32),
        mesh=mesh,
        compiler_params=pltpu.CompilerParams(needs_layout_passes=False),
        scratch_types=[
            pltpu.VMEM((2, SLAB), jnp.int32),
            pltpu.VMEM((2, CHUNK_ROWS, BATCH), jnp.float32),
            pltpu.VMEM((NHALF * NBINS * CAP * LANES,), jnp.int32),
            pltpu.VMEM((NHALF * NBINS * LANES,), jnp.int32),
            pltpu.SemaphoreType.DMA,
            pltpu.SemaphoreType.DMA,
        ],
    )(tok_flat)
    return out_t.T


def kernel(tokens, lengths):
    del lengths
    return _encode(tokens)


# speedup vs baseline: 1.0820x; 1.0154x over previous
import functools

import jax
import jax.numpy as jnp
from jax import lax
from jax.experimental import pallas as pl
from jax.experimental.pallas import tpu as pltpu
from jax.experimental.pallas import tpu_sc as plsc

VOCAB = 100000
BATCH = 1024
SEQ = 200
NTOK = BATCH * SEQ
N_WORKERS = 32
STRIPE = 3128
CHUNK_ROWS = 32
CHUNK_WORDS = CHUNK_ROWS * BATCH
TAIL_ROWS = 24
NBINS = 98
CAP = 16
NHALF = 2
LANES = 16
SLAB = 4096
N_SLABS = NTOK // SLAB
GROUPS_PER_SLAB = SLAB // LANES


def _sc_body(tok_hbm, out_hbm, slab_v, buf_v, list_v, cnt_v, dma_sem, slab_sem):
    w = lax.axis_index("s") * 2 + lax.axis_index("c")
    base = w * STRIPE
    size = jnp.minimum(STRIPE, VOCAB - base)
    base_code = base * BATCH
    lo = jnp.maximum(base_code, BATCH)
    hi = base_code + size * BATCH

    lane = lax.iota(jnp.int32, LANES)
    lane_eq = [lane == j for j in range(LANES)]
    zero_i = jnp.zeros((LANES,), jnp.int32)
    zero_f = jnp.zeros((LANES,), jnp.float32)
    one_i = jnp.ones((LANES,), jnp.int32)

    def zbuf(i, c):
        buf_v[i >> 11, (i >> 6) & 31, pl.ds((i & 63) * LANES, LANES)] = zero_f
        return c

    lax.fori_loop(0, 2 * CHUNK_WORDS // LANES, zbuf, 0)

    def zcnt(i, c):
        cnt_v[pl.ds(i * LANES, LANES)] = zero_i
        return c

    lax.fori_loop(0, NHALF * NBINS, zcnt, 0)

    def bin_one(sp, g, h):
        tok = slab_v[sp, pl.ds(g * LANES, LANES)]
        b = (g * LANES + lane) & (BATCH - 1)
        enc = (tok << 10) | b
        m = (enc >= lo) & (enc < hi)
        el = enc - base_code
        bin_ = el >> 15
        cidx = (h * NBINS + bin_) * LANES + lane
        pos = plsc.load_gather(cnt_v, [cidx], mask=m)
        inb = pos < CAP
        sm = m & inb
        plsc.store_scatter(
            list_v,
            [(h * NBINS + bin_) * (CAP * LANES) + pos * LANES + lane],
            el,
            mask=sm,
        )
        plsc.addupdate_scatter(cnt_v, [cidx], one_i, mask=sm)
        return m & ~inb

    def start_slab(s, sp):
        pltpu.async_copy(tok_hbm.at[pl.ds(s * SLAB, SLAB)], slab_v.at[sp], slab_sem)

    def wait_slab(s, sp):
        pltpu.make_async_copy(
            tok_hbm.at[pl.ds(s * SLAB, SLAB)], slab_v.at[sp], slab_sem
        ).wait()

    def scan_slab(s, ovf):
        sp = s & 1
        wait_slab(s, sp)

        @pl.when(s + 1 < N_SLABS)
        def _():
            start_slab(s + 1, 1 - sp)

        def scan_group(g, o):
            return o | bin_one(sp, 2 * g, 0) | bin_one(sp, 2 * g + 1, 1)

        return lax.fori_loop(0, GROUPS_PER_SLAB // 2, scan_group, ovf)

    start_slab(0, 0)
    ovf = lax.fori_loop(0, N_SLABS, scan_slab, lane < 0)
    any_ovf = jnp.max(ovf.astype(jnp.int32))

    n_full = size >> 5

    def scatter_chunk(c, p, value):
        val = jnp.full((LANES,), value, jnp.float32)
        pv = jnp.full((LANES,), 0, jnp.int32) + p
        chunk_code = c * CHUNK_WORDS

        for h in range(NHALF):
            cntc = cnt_v[pl.ds((h * NBINS + c) * LANES, LANES)]
            mx = jnp.max(cntc)

            def row_body(q, carry):
                row = list_v[
                    pl.ds((h * NBINS + c) * (CAP * LANES) + q * LANES, LANES)
                ]
                valid = q < cntc
                loc = row - chunk_code
                iv = loc >> 10
                ib = loc & (BATCH - 1)
                for j in range(LANES):
                    plsc.addupdate_scatter(
                        buf_v, [pv, iv, ib], val, mask=valid & lane_eq[j]
                    )
                return carry

            lax.fori_loop(0, mx, row_body, 0)

    def start_dma(c, p):
        pltpu.async_copy(
            buf_v.at[p],
            out_hbm.at[pl.ds(base + c * CHUNK_ROWS, CHUNK_ROWS)],
            dma_sem,
        )

    def wait_dma(c, p):
        pltpu.make_async_copy(
            buf_v.at[p],
            out_hbm.at[pl.ds(base + c * CHUNK_ROWS, CHUNK_ROWS)],
            dma_sem,
        ).wait()

    def tail_chunk(p):
        c = n_full
        scatter_chunk(c, p, 1.0)
        pltpu.sync_copy(
            buf_v.at[p, pl.ds(0, TAIL_ROWS)],
            out_hbm.at[pl.ds(base + c * CHUNK_ROWS, TAIL_ROWS)],
        )
        scatter_chunk(c, p, -1.0)

    @pl.when(any_ovf == 0)
    def _fast():
        def body(c, carry):
            p = c & 1

            @pl.when(c >= 2)
            def _():
                wait_dma(c - 2, p)
                scatter_chunk(c - 2, p, -1.0)

            scatter_chunk(c, p, 1.0)
            start_dma(c, p)
            return carry

        lax.fori_loop(0, n_full, body, 0)
        wait_dma(n_full - 2, n_full & 1)
        scatter_chunk(n_full - 2, n_full & 1, -1.0)
        wait_dma(n_full - 1, 1 - (n_full & 1))
        scatter_chunk(n_full - 1, 1 - (n_full & 1), -1.0)
        tail_chunk(0)

    def slow_chunk(c, rows):
        chunk_lo = jnp.maximum(base_code + c * CHUNK_WORDS, BATCH)
        chunk_hi = base_code + c * CHUNK_WORDS + rows * BATCH
        onef = jnp.full((LANES,), 1.0, jnp.float32)

        def sg(g, carry):
            tok = slab_v[0, pl.ds(g * LANES, LANES)]
            b = (g * LANES + lane) & (BATCH - 1)
            enc = (tok << 10) | b
            m = (enc >= chunk_lo) & (enc < chunk_hi)
            loc = enc - (base_code + c * CHUNK_WORDS)
            iv = jnp.where(m, loc >> 10, 0)
            ib = loc & (BATCH - 1)
            for j in range(LANES):
                plsc.addupdate_scatter(
                    buf_v, [zero_i, iv, ib], onef, mask=m & lane_eq[j]
                )
            return carry

        def ss(s, carry):
            pltpu.sync_copy(tok_hbm.at[pl.ds(s * SLAB, SLAB)], slab_v.at[0])
            return lax.fori_loop(0, GROUPS_PER_SLAB, sg, carry)

        lax.fori_loop(0, N_SLABS, ss, 0)
        pltpu.sync_copy(
            buf_v.at[0, pl.ds(0, rows)],
            out_hbm.at[pl.ds(base + c * CHUNK_ROWS, rows)],
        )

        def zb(i, carry):
            buf_v[0, i >> 6, pl.ds((i & 63) * LANES, LANES)] = zero_f
            return carry

        lax.fori_loop(0, rows * BATCH // LANES, zb, 0)

    @pl.when(any_ovf != 0)
    def _slow():
        def body(c, carry):
            slow_chunk(c, CHUNK_ROWS)
            return carry

        lax.fori_loop(0, n_full, body, 0)
        slow_chunk(n_full, TAIL_ROWS)


@jax.jit
def _encode(tokens):
    tok_flat = tokens.T.reshape(-1)
    mesh = plsc.VectorSubcoreMesh(core_axis_name="c", subcore_axis_name="s")
    out_t = pl.kernel(
        _sc_body,
        out_type=jax.ShapeDtypeStruct((VOCAB, BATCH), jnp.float32),
        mesh=mesh,
        compiler_params=pltpu.CompilerParams(needs_layout_passes=False),
        scratch_types=[
            pltpu.VMEM((2, SLAB), jnp.int32),
            pltpu.VMEM((2, CHUNK_ROWS, BATCH), jnp.float32),
            pltpu.VMEM((NHALF * NBINS * CAP * LANES,), jnp.int32),
            pltpu.VMEM((NHALF * NBINS * LANES,), jnp.int32),
            pltpu.SemaphoreType.DMA,
            pltpu.SemaphoreType.DMA,
        ],
    )(tok_flat)
    return out_t.T


def kernel(tokens, lengths):
    del lengths
    return _encode(tokens)
